# baseline (speedup 1.0000x reference)
import jax
import jax.numpy as jnp
from jax.experimental import pallas as pl

HIDDEN = 2048
EXPERTS = 64
K = 8
ROW_BLK = 512


def _gate_kernel(hs_ref, w_ref, b_ref, idx_ref, wgt_ref):
    logits = jax.lax.dot_general(
        hs_ref[...], w_ref[...],
        dimension_numbers=(((1,), (1,)), ((), ())),
        preferred_element_type=jnp.float32,
    ) + b_ref[...]

    vals = logits.T
    rows = vals.shape[1]
    eidx = jax.lax.broadcasted_iota(jnp.int32, (EXPERTS, rows), 0)
    top_vals = []
    top_idx = []
    for _ in range(K):
        m = jnp.max(vals, axis=0, keepdims=True)
        i = jnp.min(jnp.where(vals == m, eidx, EXPERTS), axis=0, keepdims=True)
        top_vals.append(m)
        top_idx.append(i)
        vals = jnp.where(eidx == i, -jnp.inf, vals)
    tv = jnp.concatenate(top_vals, axis=0)
    ti = jnp.concatenate(top_idx, axis=0)
    e = jnp.exp(tv - tv[0:1, :])
    w = e / jnp.sum(e, axis=0, keepdims=True)
    idx_ref[...] = ti
    wgt_ref[...] = w


def kernel(hidden_states, weight, bias):
    batch, seq, hidden = hidden_states.shape
    n = batch * seq
    hs = hidden_states.reshape(n, hidden)
    b = bias.reshape(1, EXPERTS)

    grid = (n // ROW_BLK,)
    idx, wgt = pl.pallas_call(
        _gate_kernel,
        grid=grid,
        in_specs=[
            pl.BlockSpec((ROW_BLK, HIDDEN), lambda i: (i, 0)),
            pl.BlockSpec((EXPERTS, HIDDEN), lambda i: (0, 0)),
            pl.BlockSpec((1, EXPERTS), lambda i: (0, 0)),
        ],
        out_specs=[
            pl.BlockSpec((K, ROW_BLK), lambda i: (0, i)),
            pl.BlockSpec((K, ROW_BLK), lambda i: (0, i)),
        ],
        out_shape=[
            jax.ShapeDtypeStruct((K, n), jnp.int32),
            jax.ShapeDtypeStruct((K, n), jnp.float32),
        ],
    )(hs, weight, b)
    return (idx.T, wgt.T)


# speedup vs baseline: 1.1763x; 1.1763x over previous
import jax
import jax.numpy as jnp
from jax.experimental import pallas as pl

HIDDEN = 2048
EXPERTS = 64
K = 8
ROW_BLK = 1024


def _gate_kernel(hs_ref, w_ref, b_ref, idx_ref, wgt_ref):
    logits = jax.lax.dot_general(
        hs_ref[...], w_ref[...],
        dimension_numbers=(((1,), (1,)), ((), ())),
        preferred_element_type=jnp.float32,
    ) + b_ref[...]

    vals = logits.T
    rows = vals.shape[1]
    eidx = jax.lax.broadcasted_iota(jnp.int32, (EXPERTS, rows), 0)
    top_vals = []
    top_idx = []
    for _ in range(K):
        m = jnp.max(vals, axis=0, keepdims=True)
        i = jnp.min(jnp.where(vals == m, eidx, EXPERTS), axis=0, keepdims=True)
        top_vals.append(m)
        top_idx.append(i)
        vals = jnp.where(eidx == i, -jnp.inf, vals)
    tv = jnp.concatenate(top_vals, axis=0)
    ti = jnp.concatenate(top_idx, axis=0)
    e = jnp.exp(tv - tv[0:1, :])
    w = e / jnp.sum(e, axis=0, keepdims=True)
    idx_ref[...] = ti
    wgt_ref[...] = w


def kernel(hidden_states, weight, bias):
    batch, seq, hidden = hidden_states.shape
    n = batch * seq
    hs = hidden_states.reshape(n, hidden)
    b = bias.reshape(1, EXPERTS)

    grid = (n // ROW_BLK,)
    idx, wgt = pl.pallas_call(
        _gate_kernel,
        grid=grid,
        in_specs=[
            pl.BlockSpec((ROW_BLK, HIDDEN), lambda i: (i, 0)),
            pl.BlockSpec((EXPERTS, HIDDEN), lambda i: (0, 0)),
            pl.BlockSpec((1, EXPERTS), lambda i: (0, 0)),
        ],
        out_specs=[
            pl.BlockSpec((K, ROW_BLK), lambda i: (0, i)),
            pl.BlockSpec((K, ROW_BLK), lambda i: (0, i)),
        ],
        out_shape=[
            jax.ShapeDtypeStruct((K, n), jnp.int32),
            jax.ShapeDtypeStruct((K, n), jnp.float32),
        ],
    )(hs, weight, b)
    return (idx.T, wgt.T)
